# baseline (speedup 1.0000x reference)
import jax
import jax.numpy as jnp
from jax import lax
from jax.experimental import pallas as pl
from jax.experimental.pallas import tpu as pltpu
from jax.experimental.pallas import tpu_sc as plsc

B = 4
SEQ = 8192
D = 1024
NC = 2
NS = 16
LANES = 16
NW = NC * NS
SPW = SEQ // NW
CH = 16
NCHUNK = SPW // CH
XSLOTS = 4
STEPS = NCHUNK * B
NPAIR = NCHUNK // 2
SPP = 2 * B
GROUPS = D // LANES
LOOKAHEAD = 3


def _body(x_hbm, idx_hbm, pe_hbm, out_hbm, idx_v, pe_v, x_v,
          sem_pe, sem_in, sem_out):
    wid = lax.axis_index("s") * NC + lax.axis_index("c")
    base = wid * SPW
    pltpu.sync_copy(idx_hbm.at[pl.ds(base, SPW)], idx_v)

    def pe_gather(c, pb):
        pltpu.async_copy(
            pe_hbm.at[idx_v.at[pl.ds(c * CH, CH)]],
            pe_v.at[pl.ds(pb * CH, CH)], sem_pe.at[pb])

    def wait_pe(pb):
        pltpu.make_async_copy(
            pe_hbm.at[pl.ds(0, CH)],
            pe_v.at[pl.ds(pb * CH, CH)], sem_pe.at[pb]).wait()

    def in_issue(c, b, k):
        pltpu.async_copy(
            x_hbm.at[b, pl.ds(base + c * CH, CH)], x_v.at[k], sem_in.at[k])

    def wait_in(k):
        pltpu.make_async_copy(
            x_hbm.at[0, pl.ds(0, CH)], x_v.at[k], sem_in.at[k]).wait()

    def out_issue(c, b, k):
        pltpu.async_copy(
            x_v.at[k], out_hbm.at[b, pl.ds(base + c * CH, CH)], sem_out.at[k])

    def wait_out(k):
        pltpu.make_async_copy(
            x_v.at[k], out_hbm.at[0, pl.ds(0, CH)], sem_out.at[k]).wait()

    pe_gather(0, 0)
    pe_gather(1, 1)
    for t in range(LOOKAHEAD):
        in_issue(t // B, t % B, t % XSLOTS)

    def chunk_pair(j, carry):
        for gg in range(2):
            c = 2 * j + gg
            for b in range(B):
                sb = 4 * gg + b
                k = sb % XSLOTS
                wait_in(k)
                if b == 0:
                    wait_pe(gg)

                def add_rows(r, _, gg=gg, k=k):
                    for g in range(GROUPS):
                        sl = pl.ds(g * LANES, LANES)
                        plsc.addupdate(x_v.at[k, r, sl],
                                       pe_v[gg * CH + r, sl])
                    return 0

                lax.fori_loop(0, CH, add_rows, 0)

                if b == B - 1:
                    @pl.when(j <= NPAIR - 2)
                    def _(c=c, gg=gg):
                        pe_gather(c + 2, gg)

                out_issue(c, b, k)

                tb = sb + LOOKAHEAD
                k2 = tb % XSLOTS
                c_off, b_t = divmod(tb, B)
                jmax = (STEPS - 1 - LOOKAHEAD - sb) // SPP

                @pl.when(j <= jmax)
                def _(j_=j, sb=sb, k2=k2, c_off=c_off, b_t=b_t):
                    if sb >= XSLOTS - LOOKAHEAD:
                        wait_out(k2)
                    else:
                        @pl.when(j_ >= 1)
                        def _():
                            wait_out(k2)
                    in_issue(2 * j_ + c_off, b_t, k2)
        return carry

    lax.fori_loop(0, NPAIR, chunk_pair, 0)

    for k in range(XSLOTS):
        wait_out(k)


@jax.jit
def _pe_add(x, indices, pe):
    mesh = plsc.VectorSubcoreMesh(core_axis_name="c", subcore_axis_name="s")
    return pl.kernel(
        _body,
        out_type=jax.ShapeDtypeStruct((B, SEQ, D), jnp.float32),
        mesh=mesh,
        scratch_types=[
            pltpu.VMEM((SPW,), jnp.int32),
            pltpu.VMEM((2 * CH, D), jnp.float32),
            pltpu.VMEM((XSLOTS, CH, D), jnp.float32),
            pltpu.SemaphoreType.DMA((2,)),
            pltpu.SemaphoreType.DMA((XSLOTS,)),
            pltpu.SemaphoreType.DMA((XSLOTS,)),
        ],
    )(x, indices, pe)


def kernel(x, indices, pe):
    return _pe_add(x, indices.astype(jnp.int32), pe)


# speedup vs baseline: 1.1715x; 1.1715x over previous
import jax
import jax.numpy as jnp
from jax import lax
from jax.experimental import pallas as pl
from jax.experimental.pallas import tpu as pltpu
from jax.experimental.pallas import tpu_sc as plsc

B = 4
SEQ = 8192
D = 1024
NC = 2
NS = 16
LANES = 16
NW = NC * NS
SPW = SEQ // NW
CH = 8
PECH = 2 * CH
NCHUNK = SPW // CH
NPE = SPW // PECH
XSLOTS = 8
STEPS = NCHUNK * B
SPP = 4 * B
NOUTER = STEPS // SPP
GROUPS = D // LANES
LOOKAHEAD = 6


def _body(x_hbm, idx_hbm, pe_hbm, out_hbm, idx_v, pe_v, x_v,
          sem_pe, sem_in, sem_out):
    wid = lax.axis_index("s") * NC + lax.axis_index("c")
    base = wid * SPW
    pltpu.sync_copy(idx_hbm.at[pl.ds(base, SPW)], idx_v)

    def pe_gather(p, pb):
        pltpu.async_copy(
            pe_hbm.at[idx_v.at[pl.ds(p * PECH, PECH)]],
            pe_v.at[pl.ds(pb * PECH, PECH)], sem_pe.at[pb])

    def wait_pe(pb):
        pltpu.make_async_copy(
            pe_hbm.at[pl.ds(0, PECH)],
            pe_v.at[pl.ds(pb * PECH, PECH)], sem_pe.at[pb]).wait()

    def in_issue(c, b, k):
        pltpu.async_copy(
            x_hbm.at[b, pl.ds(base + c * CH, CH)], x_v.at[k], sem_in.at[k])

    def wait_in(k):
        pltpu.make_async_copy(
            x_hbm.at[0, pl.ds(0, CH)], x_v.at[k], sem_in.at[k]).wait()

    def out_issue(c, b, k):
        pltpu.async_copy(
            x_v.at[k], out_hbm.at[b, pl.ds(base + c * CH, CH)], sem_out.at[k])

    def wait_out(k):
        pltpu.make_async_copy(
            x_v.at[k], out_hbm.at[0, pl.ds(0, CH)], sem_out.at[k]).wait()

    pe_gather(0, 0)
    pe_gather(1, 1)
    for t in range(LOOKAHEAD):
        in_issue(t // B, t % B, t % XSLOTS)

    def outer(j, carry):
        for jj in range(2):
            for gg in range(2):
                for b in range(B):
                    sb = 8 * jj + 4 * gg + b
                    k = sb % XSLOTS
                    wait_in(k)
                    if gg == 0 and b == 0:
                        wait_pe(jj)

                    def add_rows(r, _, jj=jj, gg=gg, k=k):
                        for g in range(GROUPS):
                            sl = pl.ds(g * LANES, LANES)
                            plsc.addupdate(x_v.at[k, r, sl],
                                           pe_v[jj * PECH + gg * CH + r, sl])
                        return 0

                    lax.fori_loop(0, CH, add_rows, 0)

                    if gg == 1 and b == B - 1:
                        @pl.when(j <= NOUTER - 2)
                        def _(j_=j, jj=jj):
                            pe_gather(2 * j_ + jj + 2, jj)

                    out_issue(4 * j + 2 * jj + gg, b, k)

                    tb = sb + LOOKAHEAD
                    k2 = tb % XSLOTS
                    c_off, b_t = divmod(tb, B)
                    jmax = (STEPS - 1 - LOOKAHEAD - sb) // SPP

                    @pl.when(j <= jmax)
                    def _(j_=j, sb=sb, k2=k2, c_off=c_off, b_t=b_t):
                        if sb >= XSLOTS - LOOKAHEAD:
                            wait_out(k2)
                        else:
                            @pl.when(j_ >= 1)
                            def _():
                                wait_out(k2)
                        in_issue(4 * j_ + c_off, b_t, k2)
        return carry

    lax.fori_loop(0, NOUTER, outer, 0)

    for k in range(XSLOTS):
        wait_out(k)


@jax.jit
def _pe_add(x, indices, pe):
    mesh = plsc.VectorSubcoreMesh(core_axis_name="c", subcore_axis_name="s")
    return pl.kernel(
        _body,
        out_type=jax.ShapeDtypeStruct((B, SEQ, D), jnp.float32),
        mesh=mesh,
        scratch_types=[
            pltpu.VMEM((SPW,), jnp.int32),
            pltpu.VMEM((2 * PECH, D), jnp.float32),
            pltpu.VMEM((XSLOTS, CH, D), jnp.float32),
            pltpu.SemaphoreType.DMA((2,)),
            pltpu.SemaphoreType.DMA((XSLOTS,)),
            pltpu.SemaphoreType.DMA((XSLOTS,)),
        ],
    )(x, indices, pe)


def kernel(x, indices, pe):
    return _pe_add(x, indices.astype(jnp.int32), pe)


# speedup vs baseline: 1.2459x; 1.0635x over previous
import jax
import jax.numpy as jnp
from jax import lax
from jax.experimental import pallas as pl
from jax.experimental.pallas import tpu as pltpu
from jax.experimental.pallas import tpu_sc as plsc

B = 4
SEQ = 8192
D = 1024
NC = 2
NS = 16
LANES = 16
NW = NC * NS
SPW = SEQ // NW
CH = 8
NCHUNK = SPW // CH
XSLOTS = 8
STEPS = NCHUNK * B
NPAIR = NCHUNK // 2
SPP = 2 * B
GROUPS = D // LANES
LOOKAHEAD = 5


def _body(x_hbm, idx_hbm, pe_hbm, out_hbm, idx_v, pe_v, x_v,
          sem_pe, sem_in, sem_out):
    wid = lax.axis_index("s") * NC + lax.axis_index("c")
    base = wid * SPW
    pltpu.sync_copy(idx_hbm.at[pl.ds(base, SPW)], idx_v)

    def pe_gather(c, pb):
        pltpu.async_copy(
            pe_hbm.at[idx_v.at[pl.ds(c * CH, CH)]],
            pe_v.at[pl.ds(pb * CH, CH)], sem_pe.at[pb])

    def wait_pe(pb):
        pltpu.make_async_copy(
            pe_hbm.at[pl.ds(0, CH)],
            pe_v.at[pl.ds(pb * CH, CH)], sem_pe.at[pb]).wait()

    def in_issue(c, b, k):
        pltpu.async_copy(
            x_hbm.at[b, pl.ds(base + c * CH, CH)], x_v.at[k], sem_in.at[k])

    def wait_in(k):
        pltpu.make_async_copy(
            x_hbm.at[0, pl.ds(0, CH)], x_v.at[k], sem_in.at[k]).wait()

    def out_issue(c, b, k):
        pltpu.async_copy(
            x_v.at[k], out_hbm.at[b, pl.ds(base + c * CH, CH)], sem_out.at[k])

    def wait_out(k):
        pltpu.make_async_copy(
            x_v.at[k], out_hbm.at[0, pl.ds(0, CH)], sem_out.at[k]).wait()

    pe_gather(0, 0)
    pe_gather(1, 1)
    for t in range(LOOKAHEAD):
        in_issue(t // B, t % B, t % XSLOTS)

    def chunk_pair(j, carry):
        for gg in range(2):
            c = 2 * j + gg
            for b in range(B):
                sb = 4 * gg + b
                k = sb % XSLOTS
                wait_in(k)
                if b == 0:
                    wait_pe(gg)

                def add_rows(r, _, gg=gg, k=k):
                    for g in range(GROUPS):
                        sl = pl.ds(g * LANES, LANES)
                        plsc.addupdate(x_v.at[k, r, sl],
                                       pe_v[gg * CH + r, sl])
                    return 0

                lax.fori_loop(0, CH, add_rows, 0)

                if b == B - 1:
                    @pl.when(j <= NPAIR - 2)
                    def _(c=c, gg=gg):
                        pe_gather(c + 2, gg)

                out_issue(c, b, k)

                tb = sb + LOOKAHEAD
                k2 = tb % XSLOTS
                c_off, b_t = divmod(tb, B)
                jmax = (STEPS - 1 - LOOKAHEAD - sb) // SPP

                @pl.when(j <= jmax)
                def _(j_=j, sb=sb, k2=k2, c_off=c_off, b_t=b_t):
                    if sb >= XSLOTS - LOOKAHEAD:
                        wait_out(k2)
                    else:
                        @pl.when(j_ >= 1)
                        def _():
                            wait_out(k2)
                    in_issue(2 * j_ + c_off, b_t, k2)
        return carry

    lax.fori_loop(0, NPAIR, chunk_pair, 0)

    for k in range(XSLOTS):
        wait_out(k)


@jax.jit
def _pe_add(x, indices, pe):
    mesh = plsc.VectorSubcoreMesh(core_axis_name="c", subcore_axis_name="s")
    return pl.kernel(
        _body,
        out_type=jax.ShapeDtypeStruct((B, SEQ, D), jnp.float32),
        mesh=mesh,
        scratch_types=[
            pltpu.VMEM((SPW,), jnp.int32),
            pltpu.VMEM((2 * CH, D), jnp.float32),
            pltpu.VMEM((XSLOTS, CH, D), jnp.float32),
            pltpu.SemaphoreType.DMA((2,)),
            pltpu.SemaphoreType.DMA((XSLOTS,)),
            pltpu.SemaphoreType.DMA((XSLOTS,)),
        ],
    )(x, indices, pe)


def kernel(x, indices, pe):
    return _pe_add(x, indices.astype(jnp.int32), pe)
